# baseline (speedup 1.0000x reference)
import functools

import jax
import jax.numpy as jnp
from jax import lax
from jax.experimental import pallas as pl
from jax.experimental.pallas import tpu as pltpu
from jax.experimental.pallas import tpu_sc as plsc

BS = 32
N = 8192
D = 256
DA = 11
K = 512
L = 16
NV = N // L
KV = K // L

_U = jnp.uint32
_I = jnp.int32


def _iota16():
    return lax.iota(_I, 16)


def _to_key(x):
    u = lax.bitcast_convert_type(x, _U)
    neg = lax.shift_right_logical(u, _U(31)) != _U(0)
    flip = jnp.where(neg, _U(0xFFFFFFFF), _U(0x80000000))
    a = u ^ flip
    k = (~a) ^ _U(0x80000000)
    return lax.bitcast_convert_type(k, _I)


def _from_key(kk):
    a = ~(lax.bitcast_convert_type(kk, _U) ^ _U(0x80000000))
    pos = lax.shift_right_logical(a, _U(31)) != _U(0)
    flip = jnp.where(pos, _U(0x80000000), _U(0xFFFFFFFF))
    return lax.bitcast_convert_type(a ^ flip, jnp.float32)


def _sort16(x):
    return lax.sort(x, dimension=0)


def _merge_sort_512(ref):
    def sort_each(v, _):
        ref[pl.ds(v * L, L)] = _sort16(ref[pl.ds(v * L, L)])
        return 0

    lax.fori_loop(0, KV, sort_each, 0, unroll=False)

    for m in (1, 2, 4, 8, 16):
        def merge_body(j, _, m=m):
            base = j * (2 * m)
            arr = [ref[pl.ds((base + i) * L, L)] for i in range(2 * m)]
            second = [lax.rev(x, (0,)) for x in reversed(arr[m:])]
            arr = arr[:m] + second
            s = m
            while s >= 1:
                for bs_ in range(0, 2 * m, 2 * s):
                    for i in range(bs_, bs_ + s):
                        lo = jnp.minimum(arr[i], arr[i + s])
                        hi = jnp.maximum(arr[i], arr[i + s])
                        arr[i], arr[i + s] = lo, hi
                s //= 2
            for i in range(2 * m):
                ref[pl.ds((base + i) * L, L)] = _sort16(arr[i])
            return 0

        lax.fori_loop(0, KV // (2 * m), merge_body, 0, unroll=False)


def _select_body(conf_hbm, anch_hbm, conf_out, idx_out, anch_out,
                 conf_v, keys_v, hist_v, keyc_v, idxc_v, sortk_v, packed_v,
                 outf_v, outi_v, aflat_v, abuf_v, sem, sem_a):
    nc = lax.axis_index("c")
    ns = lax.axis_index("s")
    w = ns * 2 + nc

    a_cp = pltpu.async_copy(anch_hbm.at[w], aflat_v, sem_a)
    pltpu.async_copy(conf_hbm.at[w], conf_v, sem).wait()

    def mk_keys(i, _):
        keys_v[pl.ds(i * L, L)] = _to_key(conf_v[pl.ds(i * L, L)])
        return 0

    lax.fori_loop(0, NV, mk_keys, 0, unroll=False)

    lane = _iota16()

    target = _I(K)
    m_below = _I(0)
    prefix = _I(0)
    for p in range(4):
        sh = 8 * (3 - p)

        def clear(i, _):
            hist_v[pl.ds(i * L, L)] = jnp.zeros((L,), _I)
            return 0

        lax.fori_loop(0, 256, clear, 0, unroll=False)

        def hist_pass(i, _, p=p, sh=sh, prefix=prefix):
            kk = keys_v[pl.ds(i * L, L)]
            dig = lax.shift_right_logical(kk, _I(sh)) & _I(0xFF)
            if p == 0:
                dig = dig ^ _I(0x80)
                msk = jnp.ones((L,), jnp.bool_)
            else:
                hi = lax.shift_right_logical(kk, _I(sh + 8))
                msk = hi == prefix
            addr = lane * _I(256) + dig
            cur = plsc.load_gather(hist_v, [addr])
            plsc.store_scatter(hist_v, [addr], cur + _I(1), mask=msk)
            return 0

        lax.fori_loop(0, NV, hist_pass, 0, unroll=False)

        def scan_chunk(c, carry):
            found, dfound, before, run = carry
            acc = jnp.zeros((L,), _I)
            for l in range(L):
                acc = acc + hist_v[pl.ds(l * 256 + c * L, L)]
            cum = plsc.cumsum(acc)
            crossed = (run + cum) >= target
            ncross = jnp.sum(jnp.where(crossed, _I(0), _I(1)))
            in_chunk = jnp.logical_and(ncross < _I(L), jnp.logical_not(found))
            dd = c * _I(L) + ncross
            excl = run + jnp.sum(jnp.where(lane < ncross, acc, _I(0)))
            dfound = jnp.where(in_chunk, dd, dfound)
            before = jnp.where(in_chunk, excl, before)
            found = jnp.logical_or(found, in_chunk)
            run = run + jnp.sum(acc)
            return found, dfound, before, run

        init = (jnp.bool_(False), _I(0), _I(0), _I(0))
        _, dsel, before, _ = lax.fori_loop(0, 16, scan_chunk, init,
                                           unroll=False)
        draw = dsel ^ _I(0x80) if p == 0 else dsel
        prefix = lax.shift_left(prefix, _I(8)) | draw
        target = target - before
        m_below = m_below + before

    thr = prefix
    need = target

    def compact(i, carry):
        off, ties = carry
        kk = keys_v[pl.ds(i * L, L)]
        lt = kk < thr
        eq = kk == thr
        eq_cum = plsc.cumsum(jnp.where(eq, _I(1), _I(0)))
        tie_take = jnp.logical_and(eq, (ties + eq_cum) <= need)
        sel = jnp.logical_or(lt, tie_take)
        sel_cum = plsc.cumsum(jnp.where(sel, _I(1), _I(0)))
        dest = off + sel_cum - _I(1)
        plsc.store_scatter(keyc_v, [dest], kk, mask=sel)
        plsc.store_scatter(idxc_v, [dest], i * _I(L) + lane, mask=sel)
        off = off + jnp.sum(jnp.where(sel, _I(1), _I(0)))
        ties = ties + jnp.sum(jnp.where(eq, _I(1), _I(0)))
        return off, ties

    lax.fori_loop(0, NV, compact, (_I(0), _I(0)), unroll=False)

    def cp(i, _):
        sortk_v[pl.ds(i * L, L)] = keyc_v[pl.ds(i * L, L)]
        return 0

    lax.fori_loop(0, KV, cp, 0, unroll=False)
    _merge_sort_512(sortk_v)

    def rank_pack(v, _):
        kk = keyc_v[pl.ds(v * L, L)]
        res = jnp.zeros((L,), _I)
        for step in (256, 128, 64, 32, 16, 8, 4, 2, 1):
            cand = res + _I(step)
            probe = plsc.load_gather(sortk_v, [cand - _I(1)])
            res = jnp.where(probe < kk, cand, res)
        packed = lax.shift_left(res, _I(13)) | idxc_v[pl.ds(v * L, L)]
        packed_v[pl.ds(v * L, L)] = packed
        return 0

    lax.fori_loop(0, KV, rank_pack, 0, unroll=False)

    _merge_sort_512(packed_v)

    def emit(v, _):
        sk = sortk_v[pl.ds(v * L, L)]
        outf_v[pl.ds(v * L, L)] = _from_key(sk)
        pk = packed_v[pl.ds(v * L, L)]
        outi_v[pl.ds(v * L, L)] = (pk & _I(0x1FFF)) + w * _I(N)
        return 0

    lax.fori_loop(0, KV, emit, 0, unroll=False)
    of_cp = pltpu.async_copy(outf_v, conf_out.at[w], sem)
    oi_cp = pltpu.async_copy(outi_v, idx_out.at[w], sem)

    a_cp.wait()

    def agather(v, _):
        pk = packed_v[pl.ds(v * L, L)]
        src = (pk & _I(0x1FFF)) * _I(DA)
        dst = (v * _I(L) + lane) * _I(DA)
        for j in range(DA):
            vals = plsc.load_gather(aflat_v, [src + _I(j)])
            plsc.store_scatter(abuf_v, [dst + _I(j)], vals)
        return 0

    lax.fori_loop(0, KV, agather, 0, unroll=False)
    ao_cp = pltpu.async_copy(abuf_v, anch_out.at[w], sem_a)
    of_cp.wait()
    oi_cp.wait()
    ao_cp.wait()


_CHUNK = 128
_NCH = K // _CHUNK


def _gather_body(idx_hbm, feat_hbm, feat_out,
                 idx_v, fbuf_v, sem_i, sem_f, sem_o):
    nc = lax.axis_index("c")
    ns = lax.axis_index("s")
    w = ns * 2 + nc

    pltpu.async_copy(idx_hbm.at[w], idx_v, sem_i).wait()

    prev = None
    for c in range(_NCH):
        cur = c % 2
        fg = pltpu.async_copy(feat_hbm.at[idx_v.at[c]], fbuf_v.at[cur],
                              sem_f)
        if prev is not None:
            prev.wait()
        fg.wait()
        fo = pltpu.async_copy(fbuf_v.at[cur],
                              feat_out.at[w, pl.ds(c * _CHUNK, _CHUNK)],
                              sem_o)
        prev = fo
    prev.wait()


@jax.jit
def kernel(confidence, instance_feature, anchor, k):
    mesh = plsc.VectorSubcoreMesh(core_axis_name="c", subcore_axis_name="s")

    params = pltpu.CompilerParams(needs_layout_passes=False)
    select = pl.kernel(
        _select_body,
        compiler_params=params,
        out_type=(
            jax.ShapeDtypeStruct((BS, K), jnp.float32),
            jax.ShapeDtypeStruct((BS, K), jnp.int32),
            jax.ShapeDtypeStruct((BS, K * DA), jnp.float32),
        ),
        mesh=mesh,
        scratch_types=[
            pltpu.VMEM((N,), jnp.float32),
            pltpu.VMEM((N,), jnp.int32),
            pltpu.VMEM((4096,), jnp.int32),
            pltpu.VMEM((K,), jnp.int32),
            pltpu.VMEM((K,), jnp.int32),
            pltpu.VMEM((K,), jnp.int32),
            pltpu.VMEM((K,), jnp.int32),
            pltpu.VMEM((K,), jnp.float32),
            pltpu.VMEM((K,), jnp.int32),
            pltpu.VMEM((N * DA,), jnp.float32),
            pltpu.VMEM((K * DA,), jnp.float32),
            pltpu.SemaphoreType.DMA,
            pltpu.SemaphoreType.DMA,
        ],
    )
    conf_sorted, flat_idx, anch_flat = select(
        confidence, anchor.reshape(BS, N * DA))

    gather = pl.kernel(
        _gather_body,
        compiler_params=params,
        out_type=jax.ShapeDtypeStruct((BS, K, D), jnp.float32),
        mesh=mesh,
        scratch_types=[
            pltpu.VMEM((_NCH, _CHUNK), jnp.int32),
            pltpu.VMEM((2, _CHUNK, D), jnp.float32),
            pltpu.SemaphoreType.DMA,
            pltpu.SemaphoreType.DMA,
            pltpu.SemaphoreType.DMA,
        ],
    )
    feat = gather(
        flat_idx.reshape(BS, _NCH, _CHUNK),
        instance_feature.reshape(BS * N, D),
    )

    conf_out = conf_sorted + (jnp.asarray(k, jnp.float32) - jnp.float32(K))
    return conf_out, feat, anch_flat.reshape(BS, K, DA)


# speedup vs baseline: 2.0706x; 2.0706x over previous
import functools

import jax
import jax.numpy as jnp
from jax import lax
from jax.experimental import pallas as pl
from jax.experimental.pallas import tpu as pltpu
from jax.experimental.pallas import tpu_sc as plsc

BS = 32
N = 8192
D = 256
DA = 11
K = 512
L = 16
NV = N // L
KV = K // L

_U = jnp.uint32
_I = jnp.int32


def _iota16():
    return lax.iota(_I, 16)


def _to_key(x):
    u = lax.bitcast_convert_type(x, _U)
    neg = lax.shift_right_logical(u, _U(31)) != _U(0)
    flip = jnp.where(neg, _U(0xFFFFFFFF), _U(0x80000000))
    a = u ^ flip
    k = (~a) ^ _U(0x80000000)
    return lax.bitcast_convert_type(k, _I)


def _from_key(kk):
    a = ~(lax.bitcast_convert_type(kk, _U) ^ _U(0x80000000))
    pos = lax.shift_right_logical(a, _U(31)) != _U(0)
    flip = jnp.where(pos, _U(0x80000000), _U(0xFFFFFFFF))
    return lax.bitcast_convert_type(a ^ flip, jnp.float32)


def _sort16(x):
    return lax.sort(x, dimension=0)


def _merge_sort_512(ref):
    def sort_each(v, _):
        ref[pl.ds(v * L, L)] = _sort16(ref[pl.ds(v * L, L)])
        return 0

    lax.fori_loop(0, KV, sort_each, 0, unroll=False)

    for m in (1, 2, 4, 8, 16):
        def merge_body(j, _, m=m):
            base = j * (2 * m)
            arr = [ref[pl.ds((base + i) * L, L)] for i in range(2 * m)]
            second = [lax.rev(x, (0,)) for x in reversed(arr[m:])]
            arr = arr[:m] + second
            s = m
            while s >= 1:
                for bs_ in range(0, 2 * m, 2 * s):
                    for i in range(bs_, bs_ + s):
                        lo = jnp.minimum(arr[i], arr[i + s])
                        hi = jnp.maximum(arr[i], arr[i + s])
                        arr[i], arr[i + s] = lo, hi
                s //= 2
            for i in range(2 * m):
                ref[pl.ds((base + i) * L, L)] = _sort16(arr[i])
            return 0

        lax.fori_loop(0, KV // (2 * m), merge_body, 0, unroll=False)


def _select_body(*refs):
    (conf_hbm, *rest) = refs
    a_hbm = rest[:DA]
    (conf_out, idx_out, anch_out,
     conf_v, keys_v, hist_v, surv_v, keyc_v, idxc_v, sortk_v,
     packed_v, outf_v, outi_v, aflat_v, abuf_v, sem, sem_a) = rest[DA:]
    nc = lax.axis_index("c")
    ns = lax.axis_index("s")
    w = ns * 2 + nc

    a_cps = [pltpu.async_copy(a_hbm[j].at[w],
                              aflat_v.at[pl.ds(j * N, N)], sem_a)
             for j in range(DA)]
    pltpu.async_copy(conf_hbm.at[w], conf_v, sem).wait()

    lane = _iota16()

    def clear(i, _):
        hist_v[pl.ds(i * L, L)] = jnp.zeros((L,), _I)
        return 0

    lax.fori_loop(0, 256, clear, 0, unroll=False)

    def key_hist(i, _):
        kk = _to_key(conf_v[pl.ds(i * L, L)])
        keys_v[pl.ds(i * L, L)] = kk
        dig = (lax.shift_right_logical(kk, _I(24)) & _I(0xFF)) ^ _I(0x80)
        addr = lane * _I(256) + dig
        cur = plsc.load_gather(hist_v, [addr])
        plsc.store_scatter(hist_v, [addr], cur + _I(1))
        return 0

    lax.fori_loop(0, NV, key_hist, 0, unroll=False)

    def hist_scan(target):
        def scan_chunk(c, carry):
            found, dfound, before, run = carry
            acc = jnp.zeros((L,), _I)
            for l in range(L):
                acc = acc + hist_v[pl.ds(l * 256 + c * L, L)]
            cum = plsc.cumsum(acc)
            crossed = (run + cum) >= target
            ncross = jnp.sum(jnp.where(crossed, _I(0), _I(1)))
            in_chunk = jnp.logical_and(ncross < _I(L), jnp.logical_not(found))
            dd = c * _I(L) + ncross
            excl = run + jnp.sum(jnp.where(lane < ncross, acc, _I(0)))
            dfound = jnp.where(in_chunk, dd, dfound)
            before = jnp.where(in_chunk, excl, before)
            found = jnp.logical_or(found, in_chunk)
            run = run + jnp.sum(acc)
            return found, dfound, before, run

        init = (jnp.bool_(False), _I(0), _I(0), _I(0))
        _, dsel, before, _ = lax.fori_loop(0, 16, scan_chunk, init,
                                           unroll=False)
        return dsel, before

    target = _I(K)
    d0, before = hist_scan(target)
    prefix = d0 ^ _I(0x80)
    target = target - before
    m_below = before

    def surv_first(i, cnt):
        kk = keys_v[pl.ds(i * L, L)]
        hit = lax.shift_right_logical(kk, _I(24)) == prefix
        cum = plsc.cumsum(jnp.where(hit, _I(1), _I(0)))
        plsc.store_scatter(surv_v, [cnt + cum - _I(1)], kk, mask=hit)
        return cnt + cum[15]

    nsurv = lax.fori_loop(0, NV, surv_first, _I(0), unroll=False)

    for p in range(1, 4):
        sh = 8 * (3 - p)
        lax.fori_loop(0, 256, clear, 0, unroll=False)
        ntrip = lax.div(nsurv + _I(L - 1), _I(L))

        def hist_pass(i, _, sh=sh, nsurv=nsurv):
            kk = surv_v[pl.ds(i * L, L)]
            valid = (i * _I(L) + lane) < nsurv
            dig = lax.shift_right_logical(kk, _I(sh)) & _I(0xFF)
            addr = lane * _I(256) + dig
            cur = plsc.load_gather(hist_v, [addr])
            plsc.store_scatter(hist_v, [addr], cur + _I(1), mask=valid)
            return 0

        lax.fori_loop(0, ntrip, hist_pass, 0, unroll=False)
        dp, before = hist_scan(target)
        target = target - before
        m_below = m_below + before
        prefix = lax.shift_left(prefix, _I(8)) | dp

        if p < 3:
            def surv_next(i, cnt, sh=sh, dp=dp, nsurv=nsurv):
                kk = surv_v[pl.ds(i * L, L)]
                valid = (i * _I(L) + lane) < nsurv
                dig = lax.shift_right_logical(kk, _I(sh)) & _I(0xFF)
                hit = jnp.logical_and(dig == dp, valid)
                cum = plsc.cumsum(jnp.where(hit, _I(1), _I(0)))
                plsc.store_scatter(surv_v, [cnt + cum - _I(1)], kk,
                                   mask=hit)
                return cnt + cum[15]

            nsurv = lax.fori_loop(0, ntrip, surv_next, _I(0), unroll=False)

    thr = prefix
    need = target

    def compact(i, carry):
        off, ties = carry
        kk = keys_v[pl.ds(i * L, L)]
        lt = kk < thr
        eq = kk == thr
        eq_cum = plsc.cumsum(jnp.where(eq, _I(1), _I(0)))
        tie_take = jnp.logical_and(eq, (ties + eq_cum) <= need)
        sel = jnp.logical_or(lt, tie_take)
        sel_cum = plsc.cumsum(jnp.where(sel, _I(1), _I(0)))
        dest = off + sel_cum - _I(1)
        plsc.store_scatter(keyc_v, [dest], kk, mask=sel)
        plsc.store_scatter(idxc_v, [dest], i * _I(L) + lane, mask=sel)
        off = off + sel_cum[15]
        ties = ties + eq_cum[15]
        return off, ties

    lax.fori_loop(0, NV, compact, (_I(0), _I(0)), unroll=False)

    def cp(i, _):
        sortk_v[pl.ds(i * L, L)] = keyc_v[pl.ds(i * L, L)]
        return 0

    lax.fori_loop(0, KV, cp, 0, unroll=False)
    _merge_sort_512(sortk_v)

    def rank_pack(v, _):
        kk = keyc_v[pl.ds(v * L, L)]
        res = jnp.zeros((L,), _I)
        for step in (256, 128, 64, 32, 16, 8, 4, 2, 1):
            cand = res + _I(step)
            probe = plsc.load_gather(sortk_v, [cand - _I(1)])
            res = jnp.where(probe < kk, cand, res)
        packed = lax.shift_left(res, _I(13)) | idxc_v[pl.ds(v * L, L)]
        packed_v[pl.ds(v * L, L)] = packed
        return 0

    lax.fori_loop(0, KV, rank_pack, 0, unroll=False)

    _merge_sort_512(packed_v)

    def emit(v, _):
        sk = sortk_v[pl.ds(v * L, L)]
        outf_v[pl.ds(v * L, L)] = _from_key(sk)
        pk = packed_v[pl.ds(v * L, L)]
        outi_v[pl.ds(v * L, L)] = (pk & _I(0x1FFF)) + w * _I(N)
        return 0

    lax.fori_loop(0, KV, emit, 0, unroll=False)
    of_cp = pltpu.async_copy(outf_v, conf_out.at[w], sem)
    oi_cp = pltpu.async_copy(outi_v, idx_out.at[w], sem)

    for cp in a_cps:
        cp.wait()

    def agather(v, _):
        pk = packed_v[pl.ds(v * L, L)]
        src = pk & _I(0x1FFF)
        dst = (v * _I(L) + lane) * _I(DA)
        for j in range(DA):
            vals = plsc.load_gather(aflat_v, [src + _I(j * N)])
            plsc.store_scatter(abuf_v, [dst + _I(j)], vals)
        return 0

    lax.fori_loop(0, KV, agather, 0, unroll=False)
    ao_cp = pltpu.async_copy(abuf_v, anch_out.at[w], sem_a)
    of_cp.wait()
    oi_cp.wait()
    ao_cp.wait()


_CHUNK = 128
_NCH = K // _CHUNK


def _gather_body(idx_hbm, feat_hbm, feat_out,
                 idx_v, fbuf_v, sem_i, sem_f, sem_o):
    nc = lax.axis_index("c")
    ns = lax.axis_index("s")
    w = ns * 2 + nc

    icps = [pltpu.async_copy(idx_hbm.at[w, pl.ds(c * _CHUNK, _CHUNK)],
                             idx_v.at[c], sem_i) for c in range(_NCH)]
    for c in icps:
        c.wait()

    gets = [None] * _NCH
    puts = [None] * _NCH
    gets[0] = pltpu.async_copy(feat_hbm.at[idx_v.at[0]], fbuf_v.at[0],
                               sem_f.at[0])
    for c in range(_NCH):
        cur = c % 2
        if c + 1 < _NCH:
            if c >= 1:
                puts[c - 1].wait()
            gets[c + 1] = pltpu.async_copy(feat_hbm.at[idx_v.at[c + 1]],
                                           fbuf_v.at[(c + 1) % 2],
                                           sem_f.at[(c + 1) % 2])
        gets[c].wait()
        puts[c] = pltpu.async_copy(fbuf_v.at[cur],
                                   feat_out.at[w, pl.ds(c * _CHUNK, _CHUNK)],
                                   sem_o.at[cur])
    puts[_NCH - 2].wait()
    puts[_NCH - 1].wait()


@jax.jit
def kernel(confidence, instance_feature, anchor, k):
    mesh = plsc.VectorSubcoreMesh(core_axis_name="c", subcore_axis_name="s")

    params = pltpu.CompilerParams(needs_layout_passes=False)
    select = pl.kernel(
        _select_body,
        compiler_params=params,
        out_type=(
            jax.ShapeDtypeStruct((BS, K), jnp.float32),
            jax.ShapeDtypeStruct((BS, K), jnp.int32),
            jax.ShapeDtypeStruct((BS, K * DA), jnp.float32),
        ),
        mesh=mesh,
        scratch_types=[
            pltpu.VMEM((N,), jnp.float32),
            pltpu.VMEM((N,), jnp.int32),
            pltpu.VMEM((4096,), jnp.int32),
            pltpu.VMEM((N,), jnp.int32),
            pltpu.VMEM((K,), jnp.int32),
            pltpu.VMEM((K,), jnp.int32),
            pltpu.VMEM((K,), jnp.int32),
            pltpu.VMEM((K,), jnp.int32),
            pltpu.VMEM((K,), jnp.float32),
            pltpu.VMEM((K,), jnp.int32),
            pltpu.VMEM((N * DA,), jnp.float32),
            pltpu.VMEM((K * DA,), jnp.float32),
            pltpu.SemaphoreType.DMA,
            pltpu.SemaphoreType.DMA,
        ],
    )
    conf_sorted, flat_idx, anch_sel = select(
        confidence, *[anchor[:, :, j] for j in range(DA)])

    gather = pl.kernel(
        _gather_body,
        compiler_params=params,
        out_type=jax.ShapeDtypeStruct((BS, K, D), jnp.float32),
        mesh=mesh,
        scratch_types=[
            pltpu.VMEM((_NCH, _CHUNK), jnp.int32),
            pltpu.VMEM((2, _CHUNK, D), jnp.float32),
            pltpu.SemaphoreType.DMA,
            pltpu.SemaphoreType.DMA((2,)),
            pltpu.SemaphoreType.DMA((2,)),
        ],
    )
    feat = gather(flat_idx, instance_feature.reshape(BS * N, D))

    conf_out = conf_sorted + (jnp.asarray(k, jnp.float32) - jnp.float32(K))
    return conf_out, feat, anch_sel.reshape(BS, K, DA)


# speedup vs baseline: 2.2320x; 1.0780x over previous
import functools

import jax
import jax.numpy as jnp
from jax import lax
from jax.experimental import pallas as pl
from jax.experimental.pallas import tpu as pltpu
from jax.experimental.pallas import tpu_sc as plsc

BS = 32
N = 8192
D = 256
DA = 11
K = 512
L = 16
NV = N // L
KV = K // L

_U = jnp.uint32
_I = jnp.int32


def _iota16():
    return lax.iota(_I, 16)


def _to_key(x):
    u = lax.bitcast_convert_type(x, _U)
    neg = lax.shift_right_logical(u, _U(31)) != _U(0)
    flip = jnp.where(neg, _U(0xFFFFFFFF), _U(0x80000000))
    a = u ^ flip
    k = (~a) ^ _U(0x80000000)
    return lax.bitcast_convert_type(k, _I)


def _from_key(kk):
    a = ~(lax.bitcast_convert_type(kk, _U) ^ _U(0x80000000))
    pos = lax.shift_right_logical(a, _U(31)) != _U(0)
    flip = jnp.where(pos, _U(0x80000000), _U(0xFFFFFFFF))
    return lax.bitcast_convert_type(a ^ flip, jnp.float32)


def _sort16(x):
    return lax.sort(x, dimension=0)


def _merge_sort_512(ref):
    def sort_each(v, _):
        ref[pl.ds(v * L, L)] = _sort16(ref[pl.ds(v * L, L)])
        return 0

    lax.fori_loop(0, KV, sort_each, 0, unroll=False)

    for m in (1, 2, 4, 8, 16):
        def merge_body(j, _, m=m):
            base = j * (2 * m)
            arr = [ref[pl.ds((base + i) * L, L)] for i in range(2 * m)]
            second = [lax.rev(x, (0,)) for x in reversed(arr[m:])]
            arr = arr[:m] + second
            s = m
            while s >= 1:
                for bs_ in range(0, 2 * m, 2 * s):
                    for i in range(bs_, bs_ + s):
                        lo = jnp.minimum(arr[i], arr[i + s])
                        hi = jnp.maximum(arr[i], arr[i + s])
                        arr[i], arr[i + s] = lo, hi
                s //= 2
            for i in range(2 * m):
                ref[pl.ds((base + i) * L, L)] = _sort16(arr[i])
            return 0

        lax.fori_loop(0, KV // (2 * m), merge_body, 0, unroll=False)


def _select_body(*refs):
    (conf_hbm, anch_hbm, conf_out, idx_out, anch_out,
     conf_v, keys_v, hist_v, surv_v, keyc_v, idxc_v, sortk_v,
     packed_v, outf_v, outi_v, aflat_v, abuf_v, sem, sem_a) = refs
    nc = lax.axis_index("c")
    ns = lax.axis_index("s")
    w = ns * 2 + nc

    a_cps = [pltpu.async_copy(anch_hbm.at[j, w],
                              aflat_v.at[pl.ds(j * N, N)], sem_a)
             for j in range(DA)]
    pltpu.async_copy(conf_hbm.at[w], conf_v, sem).wait()

    lane = _iota16()

    def clear(i, _):
        hist_v[pl.ds(i * L, L)] = jnp.zeros((L,), _I)
        return 0

    lax.fori_loop(0, 256, clear, 0, unroll=False)

    def key_hist(i, _):
        kk = _to_key(conf_v[pl.ds(i * L, L)])
        keys_v[pl.ds(i * L, L)] = kk
        dig = (lax.shift_right_logical(kk, _I(24)) & _I(0xFF)) ^ _I(0x80)
        addr = lane * _I(256) + dig
        plsc.addupdate_scatter(hist_v, [addr], jnp.full((L,), 1, _I))
        return 0

    lax.fori_loop(0, NV, key_hist, 0, unroll=False)

    def hist_scan(target):
        def scan_chunk(c, carry):
            found, dfound, before, run = carry
            acc = jnp.zeros((L,), _I)
            for l in range(L):
                acc = acc + hist_v[pl.ds(l * 256 + c * L, L)]
            cum = plsc.cumsum(acc)
            crossed = (run + cum) >= target
            ncross = jnp.sum(jnp.where(crossed, _I(0), _I(1)))
            in_chunk = jnp.logical_and(ncross < _I(L), jnp.logical_not(found))
            dd = c * _I(L) + ncross
            excl = run + jnp.sum(jnp.where(lane < ncross, acc, _I(0)))
            dfound = jnp.where(in_chunk, dd, dfound)
            before = jnp.where(in_chunk, excl, before)
            found = jnp.logical_or(found, in_chunk)
            run = run + jnp.sum(acc)
            return found, dfound, before, run

        init = (jnp.bool_(False), _I(0), _I(0), _I(0))
        _, dsel, before, _ = lax.fori_loop(0, 16, scan_chunk, init,
                                           unroll=False)
        return dsel, before

    target = _I(K)
    d0, before = hist_scan(target)
    prefix = d0 ^ _I(0x80)
    target = target - before
    m_below = before

    def surv_first(i, cnt):
        kk = keys_v[pl.ds(i * L, L)]
        hit = lax.shift_right_logical(kk, _I(24)) == prefix
        cum = plsc.cumsum(jnp.where(hit, _I(1), _I(0)))
        plsc.store_scatter(surv_v, [cnt + cum - _I(1)], kk, mask=hit)
        return cnt + cum[15]

    nsurv = lax.fori_loop(0, NV, surv_first, _I(0), unroll=False)

    for p in range(1, 4):
        sh = 8 * (3 - p)
        lax.fori_loop(0, 256, clear, 0, unroll=False)
        ntrip = lax.div(nsurv + _I(L - 1), _I(L))

        def hist_pass(i, _, sh=sh, nsurv=nsurv):
            kk = surv_v[pl.ds(i * L, L)]
            valid = (i * _I(L) + lane) < nsurv
            dig = lax.shift_right_logical(kk, _I(sh)) & _I(0xFF)
            addr = lane * _I(256) + dig
            plsc.addupdate_scatter(hist_v, [addr], jnp.full((L,), 1, _I),
                                   mask=valid)
            return 0

        lax.fori_loop(0, ntrip, hist_pass, 0, unroll=False)
        dp, before = hist_scan(target)
        target = target - before
        m_below = m_below + before
        prefix = lax.shift_left(prefix, _I(8)) | dp

        if p < 3:
            def surv_next(i, cnt, sh=sh, dp=dp, nsurv=nsurv):
                kk = surv_v[pl.ds(i * L, L)]
                valid = (i * _I(L) + lane) < nsurv
                dig = lax.shift_right_logical(kk, _I(sh)) & _I(0xFF)
                hit = jnp.logical_and(dig == dp, valid)
                cum = plsc.cumsum(jnp.where(hit, _I(1), _I(0)))
                plsc.store_scatter(surv_v, [cnt + cum - _I(1)], kk,
                                   mask=hit)
                return cnt + cum[15]

            nsurv = lax.fori_loop(0, ntrip, surv_next, _I(0), unroll=False)

    thr = prefix
    need = target

    def compact(i, carry):
        off, ties = carry
        kk = keys_v[pl.ds(i * L, L)]
        lt = kk < thr
        eq = kk == thr
        eq_cum = plsc.cumsum(jnp.where(eq, _I(1), _I(0)))
        tie_take = jnp.logical_and(eq, (ties + eq_cum) <= need)
        sel = jnp.logical_or(lt, tie_take)
        sel_cum = plsc.cumsum(jnp.where(sel, _I(1), _I(0)))
        dest = off + sel_cum - _I(1)
        plsc.store_scatter(keyc_v, [dest], kk, mask=sel)
        plsc.store_scatter(idxc_v, [dest], i * _I(L) + lane, mask=sel)
        off = off + sel_cum[15]
        ties = ties + eq_cum[15]
        return off, ties

    lax.fori_loop(0, NV, compact, (_I(0), _I(0)), unroll=False)

    def cp(i, _):
        sortk_v[pl.ds(i * L, L)] = keyc_v[pl.ds(i * L, L)]
        return 0

    lax.fori_loop(0, KV, cp, 0, unroll=False)
    _merge_sort_512(sortk_v)

    def rank_pack(v, _):
        kk = keyc_v[pl.ds(v * L, L)]
        res = jnp.zeros((L,), _I)
        for step in (256, 128, 64, 32, 16, 8, 4, 2, 1):
            cand = res + _I(step)
            probe = plsc.load_gather(sortk_v, [cand - _I(1)])
            res = jnp.where(probe < kk, cand, res)
        packed = lax.shift_left(res, _I(13)) | idxc_v[pl.ds(v * L, L)]
        packed_v[pl.ds(v * L, L)] = packed
        return 0

    lax.fori_loop(0, KV, rank_pack, 0, unroll=False)

    _merge_sort_512(packed_v)

    def emit(v, _):
        sk = sortk_v[pl.ds(v * L, L)]
        outf_v[pl.ds(v * L, L)] = _from_key(sk)
        pk = packed_v[pl.ds(v * L, L)]
        outi_v[pl.ds(v * L, L)] = (pk & _I(0x1FFF)) + w * _I(N)
        return 0

    lax.fori_loop(0, KV, emit, 0, unroll=False)
    of_cp = pltpu.async_copy(outf_v, conf_out.at[w], sem)
    oi_cp = pltpu.async_copy(outi_v, idx_out.at[w], sem)

    for cp in a_cps:
        cp.wait()

    def agather(v, _):
        pk = packed_v[pl.ds(v * L, L)]
        src = pk & _I(0x1FFF)
        dst = (v * _I(L) + lane) * _I(DA)
        for j in range(DA):
            vals = plsc.load_gather(aflat_v, [src + _I(j * N)])
            plsc.store_scatter(abuf_v, [dst + _I(j)], vals)
        return 0

    lax.fori_loop(0, KV, agather, 0, unroll=False)
    ao_cp = pltpu.async_copy(abuf_v, anch_out.at[w], sem_a)
    of_cp.wait()
    oi_cp.wait()
    ao_cp.wait()


_CHUNK = 128
_NCH = K // _CHUNK


def _gather_body(idx_hbm, feat_hbm, feat_out,
                 idx_v, fbuf_v, sem_i, sem_f, sem_o):
    nc = lax.axis_index("c")
    ns = lax.axis_index("s")
    w = ns * 2 + nc

    icps = [pltpu.async_copy(idx_hbm.at[w, pl.ds(c * _CHUNK, _CHUNK)],
                             idx_v.at[c], sem_i) for c in range(_NCH)]
    for c in icps:
        c.wait()

    gets = [None] * _NCH
    puts = [None] * _NCH
    gets[0] = pltpu.async_copy(feat_hbm.at[idx_v.at[0]], fbuf_v.at[0],
                               sem_f.at[0])
    for c in range(_NCH):
        cur = c % 2
        if c + 1 < _NCH:
            if c >= 1:
                puts[c - 1].wait()
            gets[c + 1] = pltpu.async_copy(feat_hbm.at[idx_v.at[c + 1]],
                                           fbuf_v.at[(c + 1) % 2],
                                           sem_f.at[(c + 1) % 2])
        gets[c].wait()
        puts[c] = pltpu.async_copy(fbuf_v.at[cur],
                                   feat_out.at[w, pl.ds(c * _CHUNK, _CHUNK)],
                                   sem_o.at[cur])
    puts[_NCH - 2].wait()
    puts[_NCH - 1].wait()


@jax.jit
def kernel(confidence, instance_feature, anchor, k):
    mesh = plsc.VectorSubcoreMesh(core_axis_name="c", subcore_axis_name="s")

    params = pltpu.CompilerParams(needs_layout_passes=False)
    select = pl.kernel(
        _select_body,
        compiler_params=params,
        out_type=(
            jax.ShapeDtypeStruct((BS, K), jnp.float32),
            jax.ShapeDtypeStruct((BS, K), jnp.int32),
            jax.ShapeDtypeStruct((BS, K * DA), jnp.float32),
        ),
        mesh=mesh,
        scratch_types=[
            pltpu.VMEM((N,), jnp.float32),
            pltpu.VMEM((N,), jnp.int32),
            pltpu.VMEM((4096,), jnp.int32),
            pltpu.VMEM((N,), jnp.int32),
            pltpu.VMEM((K,), jnp.int32),
            pltpu.VMEM((K,), jnp.int32),
            pltpu.VMEM((K,), jnp.int32),
            pltpu.VMEM((K,), jnp.int32),
            pltpu.VMEM((K,), jnp.float32),
            pltpu.VMEM((K,), jnp.int32),
            pltpu.VMEM((N * DA,), jnp.float32),
            pltpu.VMEM((K * DA,), jnp.float32),
            pltpu.SemaphoreType.DMA,
            pltpu.SemaphoreType.DMA,
        ],
    )
    conf_sorted, flat_idx, anch_sel = select(
        confidence, jnp.transpose(anchor, (2, 0, 1)))

    gather = pl.kernel(
        _gather_body,
        compiler_params=params,
        out_type=jax.ShapeDtypeStruct((BS, K, D), jnp.float32),
        mesh=mesh,
        scratch_types=[
            pltpu.VMEM((_NCH, _CHUNK), jnp.int32),
            pltpu.VMEM((2, _CHUNK, D), jnp.float32),
            pltpu.SemaphoreType.DMA,
            pltpu.SemaphoreType.DMA((2,)),
            pltpu.SemaphoreType.DMA((2,)),
        ],
    )
    feat = gather(flat_idx, instance_feature.reshape(BS * N, D))

    conf_out = conf_sorted + (jnp.asarray(k, jnp.float32) - jnp.float32(K))
    return conf_out, feat, anch_sel.reshape(BS, K, DA)
